# baseline (speedup 1.0000x reference)
import functools

import jax
import jax.numpy as jnp
from jax import lax
from jax.experimental import pallas as pl
from jax.experimental.pallas import tpu as pltpu
from jax.experimental.pallas import tpu_sc as plsc

NUM_SC_CORES = 2
NUM_SUBCORES = 16
NUM_WORKERS = NUM_SC_CORES * NUM_SUBCORES
ROW_PAD = 16
ROW_OUT = 12
CHUNK = 128


def _expmap_table_body(pose_ref, out_ref):
    tv = pose_ref[...].T
    tx, ty, tz = tv[0:1], tv[1:2], tv[2:3]
    ax, ay, az = tv[3:4], tv[4:5], tv[5:6]
    theta2 = ax * ax + ay * ay + az * az
    theta = jnp.sqrt(theta2)
    near = theta < 1e-2
    theta_nz = jnp.where(near, 1.0, theta)
    theta2_nz = jnp.where(near, 1.0, theta2)
    sine = jnp.sin(theta)
    cosine = jnp.where(near, 8.0 / (4.0 + theta2) - 1.0, jnp.cos(theta))
    sbt = jnp.where(near, 0.5 * cosine + 0.5, sine / theta_nz)
    omc = jnp.where(near, 0.5 * sbt, (1.0 - cosine) / theta2_nz)
    wx, wy, wz = sbt * ax, sbt * ay, sbt * az
    r00 = omc * ax * ax + cosine
    r01 = omc * ax * ay - wz
    r02 = omc * ax * az + wy
    r10 = omc * ay * ax + wz
    r11 = omc * ay * ay + cosine
    r12 = omc * ay * az - wx
    r20 = omc * az * ax - wy
    r21 = omc * az * ay + wx
    r22 = omc * az * az + cosine
    zero = jnp.zeros_like(r00)
    table_t = jnp.concatenate(
        [r00, r01, r02, tx, r10, r11, r12, ty, r20, r21, r22, tz,
         zero, zero, zero, zero], axis=0)
    out_ref[...] = table_t.T


def _expmap_table(pose):
    n = pose.shape[0]
    return pl.pallas_call(
        _expmap_table_body,
        out_shape=jax.ShapeDtypeStruct((n, ROW_PAD), jnp.float32),
    )(pose)


def _make_sc_gather(batch):
    b_per_w = batch // NUM_WORKERS
    n_chunks = b_per_w // CHUNK
    mesh = plsc.VectorSubcoreMesh(core_axis_name="c", subcore_axis_name="s")

    @functools.partial(
        pl.kernel,
        out_type=jax.ShapeDtypeStruct((batch, ROW_PAD), jnp.float32),
        mesh=mesh,
        compiler_params=pltpu.CompilerParams(use_tc_tiling_on_sc=False),
        scratch_types=[
            pltpu.VMEM((n_chunks, CHUNK), jnp.int32),
            pltpu.VMEM((b_per_w, ROW_PAD), jnp.float32),
            pltpu.SemaphoreType.DMA,
        ],
    )
    def gather(table_hbm, idx_hbm, out_hbm, idx_v, rows_v, sem):
        wid = lax.axis_index("s") * NUM_SC_CORES + lax.axis_index("c")
        pltpu.sync_copy(idx_hbm.at[pl.ds(wid * n_chunks, n_chunks)], idx_v)
        copies = [
            pltpu.async_copy(
                table_hbm.at[idx_v.at[j]],
                rows_v.at[pl.ds(j * CHUNK, CHUNK)],
                sem,
            )
            for j in range(n_chunks)
        ]
        for c in copies:
            c.wait()
        pltpu.sync_copy(rows_v, out_hbm.at[pl.ds(wid * b_per_w, b_per_w)])

    return gather


def _compact_body(in_ref, out_ref):
    x = in_ref[...]
    n = x.shape[0]
    pieces = [x[:, 16 * k:16 * k + ROW_OUT] for k in range(8)]
    out_ref[...] = jnp.stack(pieces, axis=1).reshape(8 * n, ROW_OUT)


def _compact(out128):
    n = out128.shape[0]
    blk = 256
    return pl.pallas_call(
        _compact_body,
        grid=(n // blk,),
        in_specs=[pl.BlockSpec((blk, 128), lambda i: (i, 0))],
        out_specs=pl.BlockSpec((8 * blk, ROW_OUT), lambda i: (i, 0)),
        out_shape=jax.ShapeDtypeStruct((8 * n, ROW_OUT), jnp.float32),
    )(out128)


def kernel(indices, pose_adjustment):
    batch = indices.shape[0]
    table = _expmap_table(pose_adjustment.astype(jnp.float32))
    idx = indices.astype(jnp.int32).reshape(batch // 128, 128)
    out = _make_sc_gather(batch)(table, idx)
    packed = _compact(out.reshape(batch * ROW_PAD // 128, 128))
    return packed.reshape(batch, 3, 4)


# speedup vs baseline: 1.4188x; 1.4188x over previous
import functools

import jax
import jax.numpy as jnp
from jax import lax
from jax.experimental import pallas as pl
from jax.experimental.pallas import tpu as pltpu
from jax.experimental.pallas import tpu_sc as plsc

NUM_SC_CORES = 2
NUM_SUBCORES = 16
NUM_WORKERS = NUM_SC_CORES * NUM_SUBCORES
ROW_PAD = 16
ROW_OUT = 12
CHUNK = 128


def _expmap_table_body(pose_ref, out_ref):
    tv = pose_ref[...]
    tx, ty, tz = tv[0:1], tv[1:2], tv[2:3]
    ax, ay, az = tv[3:4], tv[4:5], tv[5:6]
    theta2 = ax * ax + ay * ay + az * az
    theta = jnp.sqrt(theta2)
    near = theta < 1e-2
    theta_nz = jnp.where(near, 1.0, theta)
    theta2_nz = jnp.where(near, 1.0, theta2)
    sine = jnp.sin(theta)
    cosine = jnp.where(near, 8.0 / (4.0 + theta2) - 1.0, jnp.cos(theta))
    sbt = jnp.where(near, 0.5 * cosine + 0.5, sine / theta_nz)
    omc = jnp.where(near, 0.5 * sbt, (1.0 - cosine) / theta2_nz)
    wx, wy, wz = sbt * ax, sbt * ay, sbt * az
    r00 = omc * ax * ax + cosine
    r01 = omc * ax * ay - wz
    r02 = omc * ax * az + wy
    r10 = omc * ay * ax + wz
    r11 = omc * ay * ay + cosine
    r12 = omc * ay * az - wx
    r20 = omc * az * ax - wy
    r21 = omc * az * ay + wx
    r22 = omc * az * az + cosine
    zero = jnp.zeros_like(r00)
    out_ref[...] = jnp.concatenate(
        [r00, r01, r02, tx, r10, r11, r12, ty, r20, r21, r22, tz,
         zero, zero, zero, zero], axis=0)


def _expmap_table(pose_t):
    n = pose_t.shape[1]
    return pl.pallas_call(
        _expmap_table_body,
        out_shape=jax.ShapeDtypeStruct((ROW_PAD, n), jnp.float32),
    )(pose_t)


def _make_sc_gather(batch):
    b_per_w = batch // NUM_WORKERS
    n_chunks = b_per_w // CHUNK
    mesh = plsc.VectorSubcoreMesh(core_axis_name="c", subcore_axis_name="s")

    @functools.partial(
        pl.kernel,
        out_type=jax.ShapeDtypeStruct((batch, ROW_PAD), jnp.float32),
        mesh=mesh,
        compiler_params=pltpu.CompilerParams(use_tc_tiling_on_sc=False),
        scratch_types=[
            pltpu.VMEM((n_chunks, CHUNK), jnp.int32),
            pltpu.VMEM((b_per_w, ROW_PAD), jnp.float32),
            pltpu.SemaphoreType.DMA,
            pltpu.SemaphoreType.DMA,
        ],
    )
    def gather(table_hbm, idx_hbm, out_hbm, idx_v, rows_v, sem, sem_out):
        wid = lax.axis_index("s") * NUM_SC_CORES + lax.axis_index("c")
        base = wid * b_per_w
        pltpu.sync_copy(idx_hbm.at[pl.ds(wid * n_chunks, n_chunks)], idx_v)
        copies = [
            pltpu.async_copy(
                table_hbm.at[idx_v.at[j]],
                rows_v.at[pl.ds(j * CHUNK, CHUNK)],
                sem,
            )
            for j in range(n_chunks)
        ]
        outs = []
        for j in range(n_chunks):
            copies[j].wait()
            outs.append(
                pltpu.async_copy(
                    rows_v.at[pl.ds(j * CHUNK, CHUNK)],
                    out_hbm.at[pl.ds(base + j * CHUNK, CHUNK)],
                    sem_out,
                ))
        for o in outs:
            o.wait()

    return gather


def kernel(indices, pose_adjustment):
    batch = indices.shape[0]
    table_t = _expmap_table(pose_adjustment.T.astype(jnp.float32))
    table = table_t.T
    idx = indices.astype(jnp.int32).reshape(batch // 128, 128)
    out = _make_sc_gather(batch)(table, idx)
    return out[:, :ROW_OUT].reshape(batch, 3, 4)


# speedup vs baseline: 1.4215x; 1.0019x over previous
import functools

import jax
import jax.numpy as jnp
from jax import lax
from jax.experimental import pallas as pl
from jax.experimental.pallas import tpu as pltpu
from jax.experimental.pallas import tpu_sc as plsc

NUM_SC_CORES = 2
NUM_SUBCORES = 16
NUM_WORKERS = NUM_SC_CORES * NUM_SUBCORES
ROW_PAD = 16
ROW_OUT = 12
CHUNK = 128


def _expmap_table_body(pose_ref, out_ref):
    tv = pose_ref[...]
    tx, ty, tz = tv[0:1], tv[1:2], tv[2:3]
    ax, ay, az = tv[3:4], tv[4:5], tv[5:6]
    theta2 = ax * ax + ay * ay + az * az
    theta = jnp.sqrt(theta2)
    near = theta < 1e-2
    theta_nz = jnp.where(near, 1.0, theta)
    theta2_nz = jnp.where(near, 1.0, theta2)
    sine = jnp.sin(theta)
    cosine = jnp.where(near, 8.0 / (4.0 + theta2) - 1.0, jnp.cos(theta))
    sbt = jnp.where(near, 0.5 * cosine + 0.5, sine / theta_nz)
    omc = jnp.where(near, 0.5 * sbt, (1.0 - cosine) / theta2_nz)
    wx, wy, wz = sbt * ax, sbt * ay, sbt * az
    r00 = omc * ax * ax + cosine
    r01 = omc * ax * ay - wz
    r02 = omc * ax * az + wy
    r10 = omc * ay * ax + wz
    r11 = omc * ay * ay + cosine
    r12 = omc * ay * az - wx
    r20 = omc * az * ax - wy
    r21 = omc * az * ay + wx
    r22 = omc * az * az + cosine
    zero = jnp.zeros_like(r00)
    out_ref[...] = jnp.concatenate(
        [r00, r01, r02, tx, r10, r11, r12, ty, r20, r21, r22, tz,
         zero, zero, zero, zero], axis=0)


def _expmap_table(pose_t):
    n = pose_t.shape[1]
    return pl.pallas_call(
        _expmap_table_body,
        out_shape=jax.ShapeDtypeStruct((ROW_PAD, n), jnp.float32),
    )(pose_t)


def _make_sc_gather(batch):
    b_per_w = batch // NUM_WORKERS
    n_chunks = b_per_w // CHUNK
    mesh = plsc.VectorSubcoreMesh(core_axis_name="c", subcore_axis_name="s")

    @functools.partial(
        pl.kernel,
        out_type=jax.ShapeDtypeStruct((batch, ROW_PAD), jnp.float32),
        mesh=mesh,
        compiler_params=pltpu.CompilerParams(use_tc_tiling_on_sc=False),
        scratch_types=[
            pltpu.VMEM((n_chunks, CHUNK), jnp.int32),
            pltpu.VMEM((b_per_w, ROW_PAD), jnp.float32),
            pltpu.SemaphoreType.DMA,
            pltpu.SemaphoreType.DMA,
        ],
    )
    def gather(table_hbm, idx_hbm, out_hbm, idx_v, rows_v, sem, sem_out):
        wid = lax.axis_index("s") * NUM_SC_CORES + lax.axis_index("c")
        base = wid * b_per_w
        pltpu.sync_copy(idx_hbm.at[pl.ds(wid * n_chunks, n_chunks)], idx_v)
        copies = [
            pltpu.async_copy(
                table_hbm.at[idx_v.at[j]],
                rows_v.at[pl.ds(j * CHUNK, CHUNK)],
                sem,
            )
            for j in range(n_chunks)
        ]
        outs = []
        for j in range(n_chunks):
            copies[j].wait()
            outs.append(
                pltpu.async_copy(
                    rows_v.at[pl.ds(j * CHUNK, CHUNK)],
                    out_hbm.at[pl.ds(base + j * CHUNK, CHUNK)],
                    sem_out,
                ))
        for o in outs:
            o.wait()

    return gather


def kernel(indices, pose_adjustment):
    batch = indices.shape[0]
    table_t = _expmap_table(pose_adjustment.T.astype(jnp.float32))
    table = table_t.T
    idx = indices.astype(jnp.int32).reshape(batch // 128, 128)
    out = _make_sc_gather(batch)(table, idx)
    return out.reshape(batch, 4, 4)[:, :3, :]
